# baseline (speedup 1.0000x reference)
import functools

import jax
import jax.numpy as jnp
from jax.experimental import pallas as pl
from jax.experimental.pallas import tpu as pltpu

ALPHA = 40.0
BETA = 0.25
GAMMA = 0.75
HALPHA = ALPHA * 0.5


def _body(costh_ref, label_ref, omega_ref, out_ref):
    c = HALPHA * omega_ref[0]
    x = costh_ref[...]
    t = jnp.tanh(c - HALPHA * x)
    lbl = label_ref[...]
    cols = jax.lax.broadcasted_iota(jnp.int32, t.shape, 1)
    out_ref[0, 0, 0] = jnp.sum(t)
    out_ref[0, 0, 1] = jnp.sum(jnp.where(cols == lbl, t, 0.0))


def kernel(costh, label, omega):
    B, C = costh.shape
    BR = 1024
    n_blocks = B // BR
    label2d = label.astype(jnp.int32).reshape(B, 1)
    omega1 = omega.astype(jnp.float32).reshape(1)
    partials = pl.pallas_call(
        _body,
        grid=(n_blocks,),
        in_specs=[
            pl.BlockSpec((BR, C), lambda i: (i, 0)),
            pl.BlockSpec((BR, 1), lambda i: (i, 0)),
            pl.BlockSpec(memory_space=pltpu.SMEM),
        ],
        out_specs=pl.BlockSpec((1, 1, 2), lambda i: (i, 0, 0), memory_space=pltpu.SMEM),
        out_shape=jax.ShapeDtypeStruct((n_blocks, 1, 2), jnp.float32),
        compiler_params=pltpu.CompilerParams(
            dimension_semantics=("parallel",),
        ),
    )(costh, label2d, omega1)
    t_all = jnp.sum(partials[:, 0, 0])
    t_pos = jnp.sum(partials[:, 0, 1])
    pfa = GAMMA * 0.5 * (1.0 - t_pos / B)
    pmiss = BETA * 0.5 * ((B * (C - 1) + t_all - t_pos) / (B * (C - 1)))
    return pfa + pmiss
